# Initial kernel scaffold; baseline (speedup 1.0000x reference)
import jax
import jax.numpy as jnp
from jax.experimental import pallas as pl


def kernel(logits, labels):
    raise NotImplementedError("write your pallas kernel here")


# speedup vs baseline: 1.9924x; 1.9924x over previous
import functools
import math

import jax
import jax.numpy as jnp
from jax import lax
from jax.experimental import pallas as pl
from jax.experimental.pallas import tpu as pltpu
from jax.experimental.pallas import tpu_sc as plsc

S = 64.0
MARGIN = 0.5
SCOS = S * math.cos(MARGIN)
SSIN = S * math.sin(MARGIN)

NC = 2
NS = 16
NW = NC * NS
L = 16


def _sc_margin_body(V, logits_hbm, labels_hbm, v_hbm, lab_v, idx_v, x_v, v_v, sem):
    bpw = lab_v.shape[0]
    wid = lax.axis_index("s") * NC + lax.axis_index("c")
    base = wid * bpw
    pltpu.sync_copy(labels_hbm.at[pl.ds(base, bpw)], lab_v)
    for c in range(bpw // L):
        lab = lab_v[pl.ds(c * L, L)]
        rows = base + c * L + lax.iota(jnp.int32, L)
        idx_v[pl.ds(c * L, L)] = rows * V + jnp.maximum(lab, 0)
    pltpu.async_copy(logits_hbm.at[idx_v], x_v, sem).wait()
    for c in range(bpw // L):
        x = x_v[pl.ds(c * L, L)]
        lab = lab_v[pl.ds(c * L, L)]
        t = jnp.maximum(1.0 - x * x, 0.0)
        g = 0.5 * (1.0 + t)
        for _ in range(18):
            g = 0.5 * (g + t / g)
        y = g
        v_v[pl.ds(c * L, L)] = jnp.where(lab >= 0, SCOS * x - SSIN * y, S * x)
    pltpu.sync_copy(v_v, v_hbm.at[pl.ds(base, bpw)])


@functools.lru_cache(maxsize=None)
def _make_sc_margin(B, V):
    bpw = B // NW
    return functools.partial(
        pl.kernel,
        out_type=jax.ShapeDtypeStruct((B,), jnp.float32),
        mesh=plsc.VectorSubcoreMesh(core_axis_name="c", subcore_axis_name="s"),
        scratch_types=[
            pltpu.VMEM((bpw,), jnp.int32),
            pltpu.VMEM((bpw,), jnp.int32),
            pltpu.VMEM((bpw,), jnp.float32),
            pltpu.VMEM((bpw,), jnp.float32),
            pltpu.SemaphoreType.DMA,
        ],
    )(functools.partial(_sc_margin_body, V))


def _tc_scale_body(V, rows, lab_ref, v_ref, x_ref, o_ref):
    x = x_ref[...]
    lab = lab_ref[0, 0, :].reshape(rows, 1)
    vv = v_ref[0, 0, :].reshape(rows, 1)
    cols = lax.broadcasted_iota(jnp.int32, (rows, V), 1)
    o_ref[...] = jnp.where(cols == lab, vv, x * S)


def kernel(logits, labels):
    B, V = logits.shape
    labels = labels.astype(jnp.int32)
    v = _make_sc_margin(B, V)(logits.reshape(B * V), labels)

    rows = 8
    grid = B // rows
    return pl.pallas_call(
        functools.partial(_tc_scale_body, V, rows),
        grid=(grid,),
        in_specs=[
            pl.BlockSpec((1, 1, rows), lambda i: (i, 0, 0)),
            pl.BlockSpec((1, 1, rows), lambda i: (i, 0, 0)),
            pl.BlockSpec((rows, V), lambda i: (i, 0)),
        ],
        out_specs=pl.BlockSpec((rows, V), lambda i: (i, 0)),
        out_shape=jax.ShapeDtypeStruct((B, V), jnp.float32),
    )(labels.reshape(grid, 1, rows), v.reshape(grid, 1, rows), logits)


# speedup vs baseline: 2.0018x; 1.0047x over previous
import functools
import math

import jax
import jax.numpy as jnp
from jax import lax
from jax.experimental import pallas as pl
from jax.experimental.pallas import tpu as pltpu
from jax.experimental.pallas import tpu_sc as plsc

S = 64.0
MARGIN = 0.5
SCOS = S * math.cos(MARGIN)
SSIN = S * math.sin(MARGIN)

NC = 2
NS = 16
NW = NC * NS
L = 16


def _sc_margin_body(V, logits_hbm, labels_hbm, v_hbm, lab_v, idx_v, x_v, v_v, sem):
    bpw = lab_v.shape[0]
    wid = lax.axis_index("s") * NC + lax.axis_index("c")
    base = wid * bpw
    pltpu.sync_copy(labels_hbm.at[pl.ds(base, bpw)], lab_v)
    for c in range(bpw // L):
        lab = lab_v[pl.ds(c * L, L)]
        rows = base + c * L + lax.iota(jnp.int32, L)
        idx_v[pl.ds(c * L, L)] = rows * V + jnp.maximum(lab, 0)
    pltpu.async_copy(logits_hbm.at[idx_v], x_v, sem).wait()
    for c in range(bpw // L):
        x = x_v[pl.ds(c * L, L)]
        lab = lab_v[pl.ds(c * L, L)]
        t = jnp.maximum(1.0 - x * x, 0.0)
        g = 0.5 * (1.0 + t)
        for _ in range(18):
            g = 0.5 * (g + t / g)
        y = g
        v_v[pl.ds(c * L, L)] = jnp.where(lab >= 0, SCOS * x - SSIN * y, S * x)
    pltpu.sync_copy(v_v, v_hbm.at[pl.ds(base, bpw)])


@functools.lru_cache(maxsize=None)
def _make_sc_margin(B, V):
    bpw = B // NW
    return functools.partial(
        pl.kernel,
        out_type=jax.ShapeDtypeStruct((B,), jnp.float32),
        mesh=plsc.VectorSubcoreMesh(core_axis_name="c", subcore_axis_name="s"),
        scratch_types=[
            pltpu.VMEM((bpw,), jnp.int32),
            pltpu.VMEM((bpw,), jnp.int32),
            pltpu.VMEM((bpw,), jnp.float32),
            pltpu.VMEM((bpw,), jnp.float32),
            pltpu.SemaphoreType.DMA,
        ],
    )(functools.partial(_sc_margin_body, V))


def _tc_scale_body(V, rows, lab_ref, v_ref, x_ref, o_ref):
    x = x_ref[...]
    lab = lab_ref[0, 0, :].reshape(rows, 1)
    vv = v_ref[0, 0, :].reshape(rows, 1)
    cols = lax.broadcasted_iota(jnp.int32, (rows, V), 1)
    o_ref[...] = jnp.where(cols == lab, vv, x * S)


def kernel(logits, labels):
    B, V = logits.shape
    labels = labels.astype(jnp.int32)
    v = _make_sc_margin(B, V)(logits.reshape(B * V), labels)

    rows = 16
    grid = B // rows
    return pl.pallas_call(
        functools.partial(_tc_scale_body, V, rows),
        grid=(grid,),
        in_specs=[
            pl.BlockSpec((1, 1, rows), lambda i: (i, 0, 0)),
            pl.BlockSpec((1, 1, rows), lambda i: (i, 0, 0)),
            pl.BlockSpec((rows, V), lambda i: (i, 0)),
        ],
        out_specs=pl.BlockSpec((rows, V), lambda i: (i, 0)),
        out_shape=jax.ShapeDtypeStruct((B, V), jnp.float32),
    )(labels.reshape(grid, 1, rows), v.reshape(grid, 1, rows), logits)
